# baseline (speedup 1.0000x reference)
import jax
import jax.numpy as jnp
from jax.experimental import pallas as pl
from jax.experimental.pallas import tpu as pltpu

B, V, H, L = 2, 256, 128, 3
TI = 32
NI = V // TI
PO = H
KA = PO + TI


def _mm(a2d, w):
    return jax.lax.dot_general(a2d, w, (((1,), (0,)), ((), ())),
                               preferred_element_type=jnp.float32)


def _gnn_kernel(af_ref, sf_ref, x_ref, emb_ref, u_ref, vw_ref,
                aw_ref, bw_ref, cw_ref, x_out_ref, s_out_ref,
                x_s, p_s, q_s, v_s, agg_s, invdeg_s, adb_s, lhs_s, rhs_s):
    l = pl.program_id(0)
    b = pl.program_id(1)
    it = pl.program_id(2)
    row = it * TI

    emb2 = emb_ref[...]
    emb0 = emb2[0:1, :]
    demb = emb2[1:2, :] - emb2[0:1, :]

    @pl.when(jnp.logical_and(l == 0, jnp.logical_and(b == 0, it == 0)))
    def _init():
        af = af_ref[...]
        deg = jnp.sum(af, axis=2, keepdims=True) + 1e-6
        invdeg_s[...] = jnp.broadcast_to(1.0 / deg, (B, V, H))
        x0 = x_ref[...]
        x_s[...] = x0
        m2 = jnp.concatenate([emb0, demb], axis=0)
        for ll in range(L):
            adb_s[ll, 0:2, :] = _mm(m2, aw_ref[ll])
        r_id = jax.lax.broadcasted_iota(jnp.int32, (TI * V, TI), 0) // V
        c_id = jax.lax.broadcasted_iota(jnp.int32, (TI * V, TI), 1)
        lhs_s[:, PO:KA] = (r_id == c_id).astype(jnp.bfloat16)
        for ll in range(L):
            rhs_s[ll, 0:H, :] = aw_ref[ll].astype(jnp.bfloat16)
        x2 = x0.reshape(B * V, H)
        p_s[0] = (_mm(x2, bw_ref[0]) + adb_s[0, 0:1, :]).reshape(B, V, H)
        q_s[0] = _mm(x2, cw_ref[0]).reshape(B, V, H)
        v_s[...] = _mm(x2, vw_ref[0]).reshape(B, V, H)

    @pl.when(jnp.logical_and(l > 0, jnp.logical_and(b == 0, it == 0)))
    def _layer_boundary():
        xc = x_s[...]
        x2 = xc.reshape(B * V, H)
        pre = _mm(x2, u_ref[l - 1]).reshape(B, V, H) + agg_s[...] * invdeg_s[...]
        xn = xc + jnp.maximum(pre, 0.0)
        x_s[...] = xn
        x2n = xn.reshape(B * V, H)
        p_s[l] = (_mm(x2n, bw_ref[l]) + adb_s[l, 0:1, :]).reshape(B, V, H)
        q_s[l] = _mm(x2n, cw_ref[l]).reshape(B, V, H)
        v_s[...] = _mm(x2n, vw_ref[l]).reshape(B, V, H)

    sft = sf_ref[b, pl.ds(row, TI), :]
    sft3 = sft[:, :, None]
    demb3 = demb[None, :, :]

    def set_p_rows(ll):
        rhs_s[ll, PO:KA, :] = p_s[ll, b, pl.ds(row, TI), :].astype(jnp.bfloat16)

    def set_w(t_cur):
        lhs_s[:, 0:H] = (sft3 * t_cur).reshape(TI * V, H).astype(jnp.bfloat16)

    def build_e(ll):
        m = _mm(lhs_s[...], rhs_s[ll]).reshape(TI, V, H)
        return m + q_s[ll, b][None, :, :]

    def write_agg(e_cur):
        aft = af_ref[b, pl.ds(row, TI), :]
        vb = v_s[b]
        g = jax.nn.sigmoid(e_cur)
        agg_s[b, pl.ds(row, TI), :] = jnp.sum(
            aft[:, :, None] * g * vb[None, :, :], axis=1)

    @pl.when(l == 0)
    def _phase0():
        set_w(demb3)
        set_p_rows(0)
        write_agg(build_e(0))

    @pl.when(l == 1)
    def _phase1():
        set_w(demb3)
        set_p_rows(0)
        e0 = build_e(0)
        t0 = jnp.maximum(e0, 0.0) + demb3
        set_w(t0)
        set_p_rows(1)
        write_agg(build_e(1))

    @pl.when(l == 2)
    def _phase2():
        set_w(demb3)
        set_p_rows(0)
        e0 = build_e(0)
        t0 = jnp.maximum(e0, 0.0) + demb3
        set_w(t0)
        set_p_rows(1)
        e1 = build_e(1)
        t1 = jnp.maximum(e1, 0.0) + t0
        set_w(t1)
        set_p_rows(2)
        e2 = build_e(2)
        write_agg(e2)
        s_out_ref[0] = (emb0[None, :, :]
                        + sft3 * (jnp.maximum(e2, 0.0) + t1))

    @pl.when(jnp.logical_and(l == L - 1,
                             jnp.logical_and(b == B - 1, it == NI - 1)))
    def _finalize_x():
        xc = x_s[...]
        x2 = xc.reshape(B * V, H)
        pre = _mm(x2, u_ref[L - 1]).reshape(B, V, H) + agg_s[...] * invdeg_s[...]
        x_out_ref[...] = xc + jnp.maximum(pre, 0.0)


@jax.jit
def kernel(x, A, S, emb, U, Vw, Aw, Bw, Cw):
    af = A.astype(jnp.float32)
    sf = S.astype(jnp.float32)

    full = lambda shp: pl.BlockSpec(shp, lambda l, b, i: (0,) * len(shp))

    def s_index(l, b, i):
        bb = jnp.where(l == L - 1, b, 0)
        ii = jnp.where(l == L - 1, i, 0)
        return (bb, ii, 0, 0)

    x_out, s_out = pl.pallas_call(
        _gnn_kernel,
        grid=(L, B, NI),
        in_specs=[
            full((B, V, V)),
            full((B, V, V)),
            full((B, V, H)),
            full((2, H)),
            full((L, H, H)),
            full((L, H, H)),
            full((L, H, H)),
            full((L, H, H)),
            full((L, H, H)),
        ],
        out_specs=[
            pl.BlockSpec((B, V, H), lambda l, b, i: (0, 0, 0)),
            pl.BlockSpec((1, TI, V, H), s_index),
        ],
        out_shape=[
            jax.ShapeDtypeStruct((B, V, H), jnp.float32),
            jax.ShapeDtypeStruct((B, V, V, H), jnp.float32),
        ],
        scratch_shapes=[
            pltpu.VMEM((B, V, H), jnp.float32),
            pltpu.VMEM((L, B, V, H), jnp.float32),
            pltpu.VMEM((L, B, V, H), jnp.float32),
            pltpu.VMEM((B, V, H), jnp.float32),
            pltpu.VMEM((B, V, H), jnp.float32),
            pltpu.VMEM((B, V, H), jnp.float32),
            pltpu.VMEM((L, 8, H), jnp.float32),
            pltpu.VMEM((TI * V, KA), jnp.bfloat16),
            pltpu.VMEM((L, KA, H), jnp.bfloat16),
        ],
    )(af, sf, x, emb, U, Vw, Aw, Bw, Cw)
    return (x_out, s_out)


# speedup vs baseline: 1.8260x; 1.8260x over previous
import jax
import jax.numpy as jnp
from jax.experimental import pallas as pl
from jax.experimental.pallas import tpu as pltpu

B, V, H, L = 2, 256, 128, 3
TI = 64
NI = V // TI


def _mm(a2d, w):
    return jax.lax.dot_general(a2d, w, (((1,), (0,)), ((), ())),
                               preferred_element_type=jnp.float32)


def _mmb(a2d, w):
    return jax.lax.dot_general(a2d, w.astype(jnp.bfloat16),
                               (((1,), (0,)), ((), ())),
                               preferred_element_type=jnp.float32
                               ).astype(jnp.bfloat16)


def _gnn_kernel(af_ref, sf_ref, x_ref, emb_ref, u_ref, vw_ref, aw_ref,
                bw_ref, cw_ref, x_out_ref, s_out_ref,
                x_s, p_s, q_s, v_s, agg_s, invdeg_s, adb_s):
    l = pl.program_id(0)
    b = pl.program_id(1)
    it = pl.program_id(2)
    row = it * TI

    emb2 = emb_ref[...]
    emb0 = emb2[0:1, :]
    demb = emb2[1:2, :] - emb2[0:1, :]

    @pl.when(jnp.logical_and(l == 0, jnp.logical_and(b == 0, it == 0)))
    def _init():
        af = af_ref[...]
        deg = jnp.sum(af, axis=2, keepdims=True) + 1e-6
        invdeg_s[...] = jnp.broadcast_to(1.0 / deg, (B, V, H))
        x0 = x_ref[...]
        x_s[...] = x0
        m2 = jnp.concatenate([emb0, demb], axis=0)
        for ll in range(L):
            adb_s[ll, 0:2, :] = _mm(m2, aw_ref[ll])
        x2 = x0.reshape(B * V, H)
        p_s[0] = (_mm(x2, bw_ref[0]) + adb_s[0, 0:1, :]).reshape(B, V, H)
        q_s[0] = _mm(x2, cw_ref[0]).reshape(B, V, H)
        v_s[...] = _mm(x2, vw_ref[0]).reshape(B, V, H)

    @pl.when(jnp.logical_and(l > 0, jnp.logical_and(b == 0, it == 0)))
    def _layer_boundary():
        xc = x_s[...]
        x2 = xc.reshape(B * V, H)
        pre = _mm(x2, u_ref[l - 1]).reshape(B, V, H) + agg_s[...] * invdeg_s[...]
        xn = xc + jnp.maximum(pre, 0.0)
        x_s[...] = xn
        x2n = xn.reshape(B * V, H)
        p_s[l] = (_mm(x2n, bw_ref[l]) + adb_s[l, 0:1, :]).reshape(B, V, H)
        q_s[l] = _mm(x2n, cw_ref[l]).reshape(B, V, H)
        v_s[...] = _mm(x2n, vw_ref[l]).reshape(B, V, H)

    sft = sf_ref[b, pl.ds(row, TI), :].astype(jnp.bfloat16)
    sft3 = sft[:, :, None]

    def build_e(ll, extra):
        db_ = adb_s[ll, 1:2, :][None, :, :].astype(jnp.bfloat16)
        p_ = p_s[ll, b, pl.ds(row, TI), :][:, None, :].astype(jnp.bfloat16)
        q_ = q_s[ll, b][None, :, :].astype(jnp.bfloat16)
        m = db_ + extra if extra is not None else db_
        return p_ + q_ + sft3 * m

    def write_agg(e_cur):
        aft = af_ref[b, pl.ds(row, TI), :].astype(jnp.bfloat16)
        vb = v_s[b].astype(jnp.bfloat16)
        g = jax.nn.sigmoid(e_cur)
        agg_s[b, pl.ds(row, TI), :] = jnp.sum(
            (aft[:, :, None] * g * vb[None, :, :]).astype(jnp.float32), axis=1)

    @pl.when(l == 0)
    def _phase0():
        write_agg(build_e(0, None))

    @pl.when(l == 1)
    def _phase1():
        r0 = jnp.maximum(build_e(0, None), jnp.bfloat16(0.0))
        m1 = _mmb(r0.reshape(TI * V, H), aw_ref[1]).reshape(TI, V, H)
        write_agg(build_e(1, m1))

    @pl.when(l == 2)
    def _phase2():
        r0 = jnp.maximum(build_e(0, None), jnp.bfloat16(0.0))
        m1 = _mmb(r0.reshape(TI * V, H), aw_ref[1]).reshape(TI, V, H)
        r1 = jnp.maximum(build_e(1, m1), jnp.bfloat16(0.0))
        r01 = r0 + r1
        t2 = _mmb(r01.reshape(TI * V, H), aw_ref[2]).reshape(TI, V, H)
        e2 = build_e(2, t2)
        write_agg(e2)
        s_out_ref[0] = (emb0[None, :, :].astype(jnp.bfloat16)
                        + sft3 * (demb[None, :, :].astype(jnp.bfloat16)
                                  + r01 + jnp.maximum(e2, jnp.bfloat16(0.0)))
                        ).astype(jnp.float32)

    @pl.when(jnp.logical_and(l == L - 1,
                             jnp.logical_and(b == B - 1, it == NI - 1)))
    def _finalize_x():
        xc = x_s[...]
        x2 = xc.reshape(B * V, H)
        pre = _mm(x2, u_ref[L - 1]).reshape(B, V, H) + agg_s[...] * invdeg_s[...]
        x_out_ref[...] = xc + jnp.maximum(pre, 0.0)


@jax.jit
def kernel(x, A, S, emb, U, Vw, Aw, Bw, Cw):
    af = A.astype(jnp.float32)
    sf = S.astype(jnp.float32)

    full = lambda shp: pl.BlockSpec(shp, lambda l, b, i: (0,) * len(shp))

    def s_index(l, b, i):
        bb = jnp.where(l == L - 1, b, 0)
        ii = jnp.where(l == L - 1, i, 0)
        return (bb, ii, 0, 0)

    x_out, s_out = pl.pallas_call(
        _gnn_kernel,
        grid=(L, B, NI),
        in_specs=[
            full((B, V, V)),
            full((B, V, V)),
            full((B, V, H)),
            full((2, H)),
            full((L, H, H)),
            full((L, H, H)),
            full((L, H, H)),
            full((L, H, H)),
            full((L, H, H)),
        ],
        out_specs=[
            pl.BlockSpec((B, V, H), lambda l, b, i: (0, 0, 0)),
            pl.BlockSpec((1, TI, V, H), s_index),
        ],
        out_shape=[
            jax.ShapeDtypeStruct((B, V, H), jnp.float32),
            jax.ShapeDtypeStruct((B, V, V, H), jnp.float32),
        ],
        scratch_shapes=[
            pltpu.VMEM((B, V, H), jnp.float32),
            pltpu.VMEM((L, B, V, H), jnp.float32),
            pltpu.VMEM((L, B, V, H), jnp.float32),
            pltpu.VMEM((B, V, H), jnp.float32),
            pltpu.VMEM((B, V, H), jnp.float32),
            pltpu.VMEM((B, V, H), jnp.float32),
            pltpu.VMEM((L, 8, H), jnp.float32),
        ],
    )(af, sf, x, emb, U, Vw, Aw, Bw, Cw)
    return (x_out, s_out)


# speedup vs baseline: 1.9895x; 1.0895x over previous
import jax
import jax.numpy as jnp
from jax.experimental import pallas as pl
from jax.experimental.pallas import tpu as pltpu

B, V, H, L = 2, 256, 128, 3
TI = 64
NI = V // TI


def _mm(a2d, w):
    return jax.lax.dot_general(a2d, w, (((1,), (0,)), ((), ())),
                               preferred_element_type=jnp.float32)


def _mmb(a2d, w):
    return jax.lax.dot_general(a2d, w.astype(jnp.bfloat16),
                               (((1,), (0,)), ((), ())),
                               preferred_element_type=jnp.float32
                               ).astype(jnp.bfloat16)


def _gnn_kernel(af_ref, sf_ref, x_ref, emb_ref, u_ref, vw_ref, aw_ref,
                bw_ref, cw_ref, x_out_ref, s_out_ref,
                x_s, p_s, q_s, v_s, agg_s, invdeg_s, adb_s):
    l = pl.program_id(0)
    b = pl.program_id(1)
    it = pl.program_id(2)
    row = it * TI

    emb2 = emb_ref[...]
    emb0 = emb2[0:1, :]
    demb = emb2[1:2, :] - emb2[0:1, :]

    @pl.when(jnp.logical_and(l == 0, jnp.logical_and(b == 0, it == 0)))
    def _init():
        af = af_ref[...]
        deg = jnp.sum(af, axis=2, keepdims=True) + 1e-6
        invdeg_s[...] = jnp.broadcast_to(1.0 / deg, (B, V, H))
        x0 = x_ref[...]
        x_s[...] = x0
        m2 = jnp.concatenate([emb0, demb], axis=0)
        for ll in range(L):
            adb_s[ll, 0:2, :] = _mm(m2, aw_ref[ll])
        x2 = x0.reshape(B * V, H)
        p_s[0] = (_mm(x2, bw_ref[0]) + adb_s[0, 0:1, :]).reshape(B, V, H)
        q_s[0] = _mm(x2, cw_ref[0]).reshape(B, V, H)
        v_s[...] = _mm(x2, vw_ref[0]).reshape(B, V, H)

    @pl.when(jnp.logical_and(l > 0, jnp.logical_and(b == 0, it == 0)))
    def _layer_boundary():
        xc = x_s[...]
        x2 = xc.reshape(B * V, H)
        pre = _mm(x2, u_ref[l - 1]).reshape(B, V, H) + agg_s[...] * invdeg_s[...]
        xn = xc + jnp.maximum(pre, 0.0)
        x_s[...] = xn
        x2n = xn.reshape(B * V, H)
        p_s[l] = (_mm(x2n, bw_ref[l]) + adb_s[l, 0:1, :]).reshape(B, V, H)
        q_s[l] = _mm(x2n, cw_ref[l]).reshape(B, V, H)
        v_s[...] = _mm(x2n, vw_ref[l]).reshape(B, V, H)

    sft = sf_ref[b, pl.ds(row, TI), :].astype(jnp.bfloat16)
    sft3 = sft[:, :, None]

    def build_e(ll, extra):
        p_ = p_s[ll, b, pl.ds(row, TI), :][:, None, :].astype(jnp.bfloat16)
        q_ = q_s[ll, b][None, :, :].astype(jnp.bfloat16)
        if extra is None:
            m = adb_s[ll, 1:2, :][None, :, :].astype(jnp.bfloat16)
        else:
            m = extra
        return p_ + q_ + sft3 * m

    def write_agg(e_cur):
        aft = af_ref[b, pl.ds(row, TI), :].astype(jnp.bfloat16)
        vb = v_s[b].astype(jnp.bfloat16)
        g = jnp.tanh(e_cur * jnp.bfloat16(0.5)) + jnp.bfloat16(1.0)
        agg_s[b, pl.ds(row, TI), :] = jnp.sum(
            (aft[:, :, None] * jnp.bfloat16(0.5)) * g * vb[None, :, :],
            axis=1, dtype=jnp.float32)

    @pl.when(l == 0)
    def _phase0():
        write_agg(build_e(0, None))

    demb3 = demb.astype(jnp.bfloat16)[None, :, :]

    @pl.when(l == 1)
    def _phase1():
        rd0 = jnp.maximum(build_e(0, None), jnp.bfloat16(0.0)) + demb3
        m1 = _mmb(rd0.reshape(TI * V, H), aw_ref[1]).reshape(TI, V, H)
        write_agg(build_e(1, m1))

    @pl.when(l == 2)
    def _phase2():
        rd0 = jnp.maximum(build_e(0, None), jnp.bfloat16(0.0)) + demb3
        m1 = _mmb(rd0.reshape(TI * V, H), aw_ref[1]).reshape(TI, V, H)
        r1 = jnp.maximum(build_e(1, m1), jnp.bfloat16(0.0))
        rd01 = rd0 + r1
        t2 = _mmb(rd01.reshape(TI * V, H), aw_ref[2]).reshape(TI, V, H)
        e2 = build_e(2, t2)
        write_agg(e2)
        s_out_ref[0] = (emb0[None, :, :].astype(jnp.bfloat16)
                        + sft3 * (rd01 + jnp.maximum(e2, jnp.bfloat16(0.0)))
                        ).astype(jnp.float32)

    @pl.when(jnp.logical_and(l == L - 1,
                             jnp.logical_and(b == B - 1, it == NI - 1)))
    def _finalize_x():
        xc = x_s[...]
        x2 = xc.reshape(B * V, H)
        pre = _mm(x2, u_ref[L - 1]).reshape(B, V, H) + agg_s[...] * invdeg_s[...]
        x_out_ref[...] = xc + jnp.maximum(pre, 0.0)


@jax.jit
def kernel(x, A, S, emb, U, Vw, Aw, Bw, Cw):
    af = A.astype(jnp.float32)
    sf = S.astype(jnp.float32)

    full = lambda shp: pl.BlockSpec(shp, lambda l, b, i: (0,) * len(shp))

    def s_index(l, b, i):
        bb = jnp.where(l == L - 1, b, 0)
        ii = jnp.where(l == L - 1, i, 0)
        return (bb, ii, 0, 0)

    x_out, s_out = pl.pallas_call(
        _gnn_kernel,
        grid=(L, B, NI),
        in_specs=[
            full((B, V, V)),
            full((B, V, V)),
            full((B, V, H)),
            full((2, H)),
            full((L, H, H)),
            full((L, H, H)),
            full((L, H, H)),
            full((L, H, H)),
            full((L, H, H)),
        ],
        out_specs=[
            pl.BlockSpec((B, V, H), lambda l, b, i: (0, 0, 0)),
            pl.BlockSpec((1, TI, V, H), s_index),
        ],
        out_shape=[
            jax.ShapeDtypeStruct((B, V, H), jnp.float32),
            jax.ShapeDtypeStruct((B, V, V, H), jnp.float32),
        ],
        scratch_shapes=[
            pltpu.VMEM((B, V, H), jnp.float32),
            pltpu.VMEM((L, B, V, H), jnp.float32),
            pltpu.VMEM((L, B, V, H), jnp.float32),
            pltpu.VMEM((B, V, H), jnp.float32),
            pltpu.VMEM((B, V, H), jnp.float32),
            pltpu.VMEM((B, V, H), jnp.float32),
            pltpu.VMEM((L, 8, H), jnp.float32),
        ],
    )(af, sf, x, emb, U, Vw, Aw, Bw, Cw)
    return (x_out, s_out)


# speedup vs baseline: 2.0778x; 1.0444x over previous
import jax
import jax.numpy as jnp
from jax.experimental import pallas as pl
from jax.experimental.pallas import tpu as pltpu

B, V, H, L = 2, 256, 128, 3
TI = 64
NI = V // TI


def _mm(a2d, w):
    return jax.lax.dot_general(a2d, w, (((1,), (0,)), ((), ())),
                               preferred_element_type=jnp.float32)


def _mmb(a2d, w):
    return jax.lax.dot_general(a2d, w.astype(jnp.bfloat16),
                               (((1,), (0,)), ((), ())),
                               preferred_element_type=jnp.float32
                               ).astype(jnp.bfloat16)


def _gnn_kernel(af_ref, sf_ref, x_ref, emb_ref, u_ref, vw_ref, aw_ref,
                bw_ref, cw_ref, x_out_ref, s_out_ref,
                x_s, p_s, q_s, v_s, agg_s, invdeg_s, adb_s):
    l = pl.program_id(0)
    b = pl.program_id(1)
    it = pl.program_id(2)
    row = it * TI

    emb2 = emb_ref[...]
    emb0 = emb2[0:1, :]
    demb = emb2[1:2, :] - emb2[0:1, :]

    @pl.when(jnp.logical_and(l == 0, jnp.logical_and(b == 0, it == 0)))
    def _init():
        af = af_ref[...]
        deg = jnp.sum(af, axis=2, keepdims=True) + 1e-6
        invdeg_s[...] = jnp.broadcast_to(1.0 / deg, (B, V, H))
        x0 = x_ref[...]
        x_s[...] = x0
        m2 = jnp.concatenate([emb0, demb], axis=0)
        for ll in range(L):
            adb_s[ll, 0:2, :] = _mm(m2, aw_ref[ll])
        x2 = x0.reshape(B * V, H)
        p_s[0] = (_mm(x2, bw_ref[0]) + adb_s[0, 0:1, :]).reshape(B, V, H)
        q_s[0] = _mm(x2, cw_ref[0]).reshape(B, V, H)
        v_s[...] = _mm(x2, vw_ref[0]).reshape(B, V, H)

    @pl.when(jnp.logical_and(l > 0, jnp.logical_and(b == 0, it == 0)))
    def _layer_boundary():
        xc = x_s[...]
        x2 = xc.reshape(B * V, H)
        pre = _mm(x2, u_ref[l - 1]).reshape(B, V, H) + agg_s[...] * invdeg_s[...]
        xn = xc + jnp.maximum(pre, 0.0)
        x_s[...] = xn
        x2n = xn.reshape(B * V, H)
        p_s[l] = (_mm(x2n, bw_ref[l]) + adb_s[l, 0:1, :]).reshape(B, V, H)
        q_s[l] = _mm(x2n, cw_ref[l]).reshape(B, V, H)
        v_s[...] = _mm(x2n, vw_ref[l]).reshape(B, V, H)

    sft = sf_ref[b, pl.ds(row, TI), :].astype(jnp.bfloat16)
    sft3 = sft[:, :, None]

    def build_e(ll, extra):
        p_ = p_s[ll, b, pl.ds(row, TI), :][:, None, :].astype(jnp.bfloat16)
        q_ = q_s[ll, b][None, :, :].astype(jnp.bfloat16)
        if extra is None:
            m = adb_s[ll, 1:2, :][None, :, :].astype(jnp.bfloat16)
        else:
            m = extra
        return p_ + q_ + sft3 * m

    def write_agg(e_cur):
        aft = af_ref[b, pl.ds(row, TI), :].astype(jnp.bfloat16)
        bias3 = ((aft - jnp.bfloat16(1.0)) * jnp.bfloat16(1000.0))[:, :, None]
        vbh = v_s[b].astype(jnp.bfloat16) * jnp.bfloat16(0.5)
        tm = jnp.tanh(e_cur * jnp.bfloat16(0.5) + bias3)
        agg_s[b, pl.ds(row, TI), :] = (
            jnp.sum(tm * vbh[None, :, :], axis=1, dtype=jnp.float32)
            + jnp.sum(vbh, axis=0, dtype=jnp.float32)[None, :])

    @pl.when(l == 0)
    def _phase0():
        write_agg(build_e(0, None))

    demb3 = demb.astype(jnp.bfloat16)[None, :, :]

    @pl.when(l == 1)
    def _phase1():
        rd0 = jnp.maximum(build_e(0, None), jnp.bfloat16(0.0)) + demb3
        m1 = _mmb(rd0.reshape(TI * V, H), aw_ref[1]).reshape(TI, V, H)
        write_agg(build_e(1, m1))

    @pl.when(l == 2)
    def _phase2():
        rd0 = jnp.maximum(build_e(0, None), jnp.bfloat16(0.0)) + demb3
        m1 = _mmb(rd0.reshape(TI * V, H), aw_ref[1]).reshape(TI, V, H)
        r1 = jnp.maximum(build_e(1, m1), jnp.bfloat16(0.0))
        rd01 = rd0 + r1
        t2 = _mmb(rd01.reshape(TI * V, H), aw_ref[2]).reshape(TI, V, H)
        e2 = build_e(2, t2)
        write_agg(e2)
        s_out_ref[0] = (emb0[None, :, :].astype(jnp.bfloat16)
                        + sft3 * (rd01 + jnp.maximum(e2, jnp.bfloat16(0.0)))
                        ).astype(jnp.float32)

    @pl.when(jnp.logical_and(l == L - 1,
                             jnp.logical_and(b == B - 1, it == NI - 1)))
    def _finalize_x():
        xc = x_s[...]
        x2 = xc.reshape(B * V, H)
        pre = _mm(x2, u_ref[L - 1]).reshape(B, V, H) + agg_s[...] * invdeg_s[...]
        x_out_ref[...] = xc + jnp.maximum(pre, 0.0)


@jax.jit
def kernel(x, A, S, emb, U, Vw, Aw, Bw, Cw):
    af = A.astype(jnp.float32)
    sf = S.astype(jnp.float32)

    full = lambda shp: pl.BlockSpec(shp, lambda l, b, i: (0,) * len(shp))

    def s_index(l, b, i):
        bb = jnp.where(l == L - 1, b, 0)
        ii = jnp.where(l == L - 1, i, 0)
        return (bb, ii, 0, 0)

    x_out, s_out = pl.pallas_call(
        _gnn_kernel,
        grid=(L, B, NI),
        in_specs=[
            full((B, V, V)),
            full((B, V, V)),
            full((B, V, H)),
            full((2, H)),
            full((L, H, H)),
            full((L, H, H)),
            full((L, H, H)),
            full((L, H, H)),
            full((L, H, H)),
        ],
        out_specs=[
            pl.BlockSpec((B, V, H), lambda l, b, i: (0, 0, 0)),
            pl.BlockSpec((1, TI, V, H), s_index),
        ],
        out_shape=[
            jax.ShapeDtypeStruct((B, V, H), jnp.float32),
            jax.ShapeDtypeStruct((B, V, V, H), jnp.float32),
        ],
        scratch_shapes=[
            pltpu.VMEM((B, V, H), jnp.float32),
            pltpu.VMEM((L, B, V, H), jnp.float32),
            pltpu.VMEM((L, B, V, H), jnp.float32),
            pltpu.VMEM((B, V, H), jnp.float32),
            pltpu.VMEM((B, V, H), jnp.float32),
            pltpu.VMEM((B, V, H), jnp.float32),
            pltpu.VMEM((L, 8, H), jnp.float32),
        ],
    )(af, sf, x, emb, U, Vw, Aw, Bw, Cw)
    return (x_out, s_out)
